# Initial kernel scaffold; baseline (speedup 1.0000x reference)
import jax
import jax.numpy as jnp
from jax.experimental import pallas as pl


def kernel(x, x_en, edge_index, x_idx, length, params):
    raise NotImplementedError("write your pallas kernel here")


# speedup vs baseline: 1.9983x; 1.9983x over previous
import jax
import jax.numpy as jnp
from jax.experimental import pallas as pl

B, M, K_NN = 2, 4096, 16
IN_DIM = 3
FEAT_DIMS = [64, 64, 64]
N = B * M

_KNN_R = 256


def _knn_body(hr_ref, hc_ref, out_ref):
    hr = hr_ref[0]
    hc = hc_ref[0]
    x2r = jnp.sum(hr * hr, axis=1, keepdims=True)
    x2c = jnp.sum(hc * hc, axis=1).reshape(1, M)
    d = x2r + x2c - 2.0 * jax.lax.dot_general(
        hr, hc, dimension_numbers=(((1,), (1,)), ((), ())),
        preferred_element_type=jnp.float32)
    iota = jax.lax.broadcasted_iota(jnp.int32, (_KNN_R, M), 1)
    cols = []
    for _ in range(K_NN):
        m = jnp.min(d, axis=1, keepdims=True)
        idx = jnp.min(jnp.where(d == m, iota, M), axis=1)
        cols.append(idx)
        d = jnp.where(iota == idx[:, None], jnp.inf, d)
    out_ref[0] = jnp.stack(cols, axis=1)


def _knn_pallas(h):
    D = h.shape[-1]
    grid = (B, M // _KNN_R)
    return pl.pallas_call(
        _knn_body,
        grid=grid,
        in_specs=[
            pl.BlockSpec((1, _KNN_R, D), lambda b, r: (b, r, 0)),
            pl.BlockSpec((1, M, D), lambda b, r: (b, 0, 0)),
        ],
        out_specs=pl.BlockSpec((1, _KNN_R, K_NN), lambda b, r: (b, r, 0)),
        out_shape=jax.ShapeDtypeStruct((B, M, K_NN), jnp.int32),
    )(h, h)


def _lrelu(x, s):
    return jnp.where(x >= 0, x, s * x)


def _mlp(x, layers, slope):
    for i, (W, b) in enumerate(layers):
        x = x @ W.T + b
        if i < len(layers) - 1:
            x = _lrelu(x, slope)
    return x


def _knn_edges_pallas(h, k):
    idx = _knn_pallas(h)
    nbr = idx[:, :, 1:]
    b, m, km1 = nbr.shape
    offs = (jnp.arange(b) * m)[:, None, None]
    src = (nbr + offs).reshape(-1)
    dst = (jnp.broadcast_to(jnp.arange(m)[None, :, None], (b, m, km1)) + offs).reshape(-1)
    return src, dst


def _edge_conv(hf, enf, src, dst, p, n):
    tW, tb = p['theta']
    pW, pb = p['phi']
    e = (hf[dst] - hf[src]) @ tW.T + tb + hf[src] @ pW.T + pb
    e_en = _mlp(enf[dst] - enf[src], p['ten'], -0.5) + _mlp(enf[src], p['pen'], -0.5)
    mu = jnp.mean(e, 0)
    var = jnp.var(e, 0)
    e = (e - mu) / jnp.sqrt(var + 1e-5) * p['bn_g'] + p['bn_b']
    deg = jnp.zeros((n,), e.dtype).at[dst].add(1.0)
    hmax = jax.ops.segment_max(e, dst, num_segments=n)
    hmax = jnp.where(deg[:, None] > 0, hmax, 0.0)
    emean = jax.ops.segment_sum(e_en, dst, num_segments=n) / jnp.maximum(deg, 1.0)[:, None]
    return hmax, emean


def kernel(x, x_en, edge_index, x_idx, length, params):
    h, h_en = x, x_en
    hs = []
    for i in range(len(FEAT_DIMS)):
        if i == 0:
            src, dst = edge_index[0], edge_index[1]
        else:
            src, dst = _knn_edges_pallas(h, K_NN)
        hf = h.reshape(N, -1)
        enf = h_en.reshape(N, -1)
        hm, em = _edge_conv(hf, enf, src, dst, params['layers'][i], N)
        h = _lrelu(hm, 0.2).reshape(B, M, -1)
        h_en = em.reshape(B, M, -1)
        hs.append(h_en)
    hc = jnp.concatenate(hs, 2).reshape(N, -1)
    out = _mlp(hc, params['proj'], -0.8)
    out_flat = out.reshape((N,))
    imgs = []
    off = jnp.zeros((), jnp.int32)
    for b in range(length.shape[0]):
        L = length[b]
        e = jax.lax.dynamic_slice_in_dim(out_flat, off, M)
        idx = jax.lax.dynamic_slice(x_idx, (off, jnp.zeros((), jnp.int32)), (M, 3))
        valid = jnp.arange(M) < L
        i0 = jnp.where(valid, idx[:, 0], 7)
        img = jnp.zeros((7, 64, 64), out.dtype).at[i0, idx[:, 1], idx[:, 2]].set(e)
        imgs.append(img[None])
        off = off + L
    oi = jnp.concatenate(imgs, 0)
    return oi[:, 1:, :, :]


# speedup vs baseline: 3.5353x; 1.7691x over previous
import functools

import jax
import jax.numpy as jnp
from jax import lax
from jax.experimental import pallas as pl

B, M, K_NN = 2, 4096, 16
IN_DIM = 3
N = B * M
KM1 = K_NN - 1
E_KNN = N * KM1
E0 = N * K_NN
TBL_W = 80

_KNN_R = 256
_RT = 512
_ET = 8192


def _lrelu(x, s):
    return jnp.where(x >= 0, x, s * x)


def _knn_body(hr_ref, hc_ref, out_ref):
    hr = hr_ref[0]
    hc = hc_ref[0]
    x2r = jnp.sum(hr * hr, axis=1, keepdims=True)
    x2c = jnp.sum(hc * hc, axis=1).reshape(1, M)
    d = x2r + x2c - 2.0 * lax.dot_general(
        hr, hc, dimension_numbers=(((1,), (1,)), ((), ())),
        preferred_element_type=jnp.float32)
    iota = lax.broadcasted_iota(jnp.int32, (_KNN_R, M), 1)
    cols = []
    for _ in range(K_NN):
        m = jnp.min(d, axis=1, keepdims=True)
        idx = jnp.min(jnp.where(d == m, iota, M), axis=1)
        cols.append(idx)
        d = jnp.where(iota == idx[:, None], jnp.inf, d)
    out_ref[0] = jnp.stack(cols, axis=1) + pl.program_id(0) * M


def _knn_pallas(h):
    D = h.shape[-1]
    return pl.pallas_call(
        _knn_body,
        grid=(B, M // _KNN_R),
        in_specs=[
            pl.BlockSpec((1, _KNN_R, D), lambda b, r: (b, r, 0)),
            pl.BlockSpec((1, M, D), lambda b, r: (b, 0, 0)),
        ],
        out_specs=pl.BlockSpec((1, _KNN_R, K_NN), lambda b, r: (b, r, 0)),
        out_shape=jax.ShapeDtypeStruct((B, M, K_NN), jnp.int32),
    )(h, h)


def _prep_body(h_ref, en_ref, wp_ref, bias_ref, wq_ref, p_ref, tab_ref):
    h = h_ref[...]
    p_ref[...] = jnp.dot(h, wp_ref[...], preferred_element_type=jnp.float32) + bias_ref[...]
    q = jnp.dot(h, wq_ref[...], preferred_element_type=jnp.float32)
    pad = jnp.zeros((h.shape[0], TBL_W - 65), jnp.float32)
    tab_ref[...] = jnp.concatenate([q, en_ref[...], pad], axis=1)


def _prep(hflat, enflat, wp, bias, wq):
    d = hflat.shape[1]
    return pl.pallas_call(
        _prep_body,
        grid=(N // _RT,),
        in_specs=[
            pl.BlockSpec((_RT, d), lambda i: (i, 0)),
            pl.BlockSpec((_RT, 1), lambda i: (i, 0)),
            pl.BlockSpec((d, 64), lambda i: (0, 0)),
            pl.BlockSpec((1, 64), lambda i: (0, 0)),
            pl.BlockSpec((d, 64), lambda i: (0, 0)),
        ],
        out_specs=[
            pl.BlockSpec((_RT, 64), lambda i: (i, 0)),
            pl.BlockSpec((_RT, TBL_W), lambda i: (i, 0)),
        ],
        out_shape=[
            jax.ShapeDtypeStruct((N, 64), jnp.float32),
            jax.ShapeDtypeStruct((N, TBL_W), jnp.float32),
        ],
    )(hflat, enflat, wp, bias, wq)


def _stats_body(p_ref, rows_ref, g_ref, mu_ref, s_ref, mx_ref, mn_ref):
    i = pl.program_id(0)
    p = p_ref[...]
    q = rows_ref[..., :64]
    s1 = jnp.sum(q, axis=1)
    s2 = jnp.sum(q * q, axis=1)
    mx_ref[...] = jnp.max(q, axis=1)
    mn_ref[...] = jnp.min(q, axis=1)

    @pl.when(i == 0)
    def _():
        mu_ref[...] = jnp.zeros_like(mu_ref)
        s_ref[...] = jnp.zeros_like(s_ref)

    mu_ref[...] += jnp.sum(KM1 * p + s1, axis=0, keepdims=True)
    s_ref[...] += jnp.sum(KM1 * p * p + 2.0 * p * s1 + s2, axis=0, keepdims=True)

    @pl.when(i == pl.num_programs(0) - 1)
    def _():
        esz = jnp.float32(E_KNN)
        mu = mu_ref[...] / esz
        var = s_ref[...] / esz - mu * mu
        mu_ref[...] = mu
        s_ref[...] = g_ref[...] * lax.rsqrt(var + 1e-5)


def _stats(p, rows3, g):
    return pl.pallas_call(
        _stats_body,
        grid=(N // _RT,),
        in_specs=[
            pl.BlockSpec((_RT, 64), lambda i: (i, 0)),
            pl.BlockSpec((_RT, KM1, TBL_W), lambda i: (i, 0, 0)),
            pl.BlockSpec((1, 64), lambda i: (0, 0)),
        ],
        out_specs=[
            pl.BlockSpec((1, 64), lambda i: (0, 0)),
            pl.BlockSpec((1, 64), lambda i: (0, 0)),
            pl.BlockSpec((_RT, 64), lambda i: (i, 0)),
            pl.BlockSpec((_RT, 64), lambda i: (i, 0)),
        ],
        out_shape=[
            jax.ShapeDtypeStruct((1, 64), jnp.float32),
            jax.ShapeDtypeStruct((1, 64), jnp.float32),
            jax.ShapeDtypeStruct((N, 64), jnp.float32),
            jax.ShapeDtypeStruct((N, 64), jnp.float32),
        ],
    )(p, rows3, g)


def _finish_body(p_ref, mx_ref, mn_ref, een_ref, mu_ref, s_ref, beta_ref,
                 h_ref, hen_ref):
    s = s_ref[...]
    mq = jnp.where(s >= 0, mx_ref[...], mn_ref[...])
    hm = (p_ref[...] + mq - mu_ref[...]) * s + beta_ref[...]
    h_ref[...] = _lrelu(hm, 0.2)
    hen_ref[...] = jnp.mean(een_ref[...], axis=1, keepdims=True)


def _finish(p, mx, mn, een_r, mu, s, beta):
    return pl.pallas_call(
        _finish_body,
        grid=(N // _RT,),
        in_specs=[
            pl.BlockSpec((_RT, 64), lambda i: (i, 0)),
            pl.BlockSpec((_RT, 64), lambda i: (i, 0)),
            pl.BlockSpec((_RT, 64), lambda i: (i, 0)),
            pl.BlockSpec((_RT, KM1), lambda i: (i, 0)),
            pl.BlockSpec((1, 64), lambda i: (0, 0)),
            pl.BlockSpec((1, 64), lambda i: (0, 0)),
            pl.BlockSpec((1, 64), lambda i: (0, 0)),
        ],
        out_specs=[
            pl.BlockSpec((_RT, 64), lambda i: (i, 0)),
            pl.BlockSpec((_RT, 1), lambda i: (i, 0)),
        ],
        out_shape=[
            jax.ShapeDtypeStruct((N, 64), jnp.float32),
            jax.ShapeDtypeStruct((N, 1), jnp.float32),
        ],
    )(p, mx, mn, een_r, mu, s, beta)


def _edge_mlp_body(xd_ref, xs_ref, *refs):
    wrefs, out_ref = refs[:-1], refs[-1]
    w = [r[...] for r in wrefs]

    def mlp(x, ws):
        w1, b1, w2, b2, w3, b3, w4, b4 = ws
        y = _lrelu(x * w1 + b1, -0.5)
        y = _lrelu(jnp.dot(y, w2, preferred_element_type=jnp.float32) + b2, -0.5)
        y = _lrelu(jnp.dot(y, w3, preferred_element_type=jnp.float32) + b3, -0.5)
        return jnp.dot(y, w4, preferred_element_type=jnp.float32) + b4

    out_ref[...] = mlp(xd_ref[...], w[:8]) + mlp(xs_ref[...], w[8:])


def _edge_mlp(xd, xs, ten_t, pen_t):
    e = xd.shape[0]
    wspecs, wargs = [], []
    for (w, b) in ten_t + pen_t:
        wspecs += [pl.BlockSpec(w.shape, lambda i: (0, 0)),
                   pl.BlockSpec(b.shape, lambda i: (0, 0))]
        wargs += [w, b]
    return pl.pallas_call(
        _edge_mlp_body,
        grid=(e // _ET,),
        in_specs=[pl.BlockSpec((_ET, 1), lambda i: (i, 0)),
                  pl.BlockSpec((_ET, 1), lambda i: (i, 0))] + wspecs,
        out_specs=pl.BlockSpec((_ET, 1), lambda i: (i, 0)),
        out_shape=jax.ShapeDtypeStruct((e, 1), jnp.float32),
    )(xd, xs, *wargs)


def _l0prep_body(h_ref, wp_ref, bias_ref, wq_ref, a_ref, b_ref):
    h = h_ref[...]
    a_ref[...] = jnp.dot(h, wp_ref[...], preferred_element_type=jnp.float32) + bias_ref[...]
    b_ref[...] = jnp.dot(h, wq_ref[...], preferred_element_type=jnp.float32)


def _l0prep(hflat, wp, bias, wq):
    d = hflat.shape[1]
    return pl.pallas_call(
        _l0prep_body,
        grid=(N // _RT,),
        in_specs=[
            pl.BlockSpec((_RT, d), lambda i: (i, 0)),
            pl.BlockSpec((d, 64), lambda i: (0, 0)),
            pl.BlockSpec((1, 64), lambda i: (0, 0)),
            pl.BlockSpec((d, 64), lambda i: (0, 0)),
        ],
        out_specs=[
            pl.BlockSpec((_RT, 64), lambda i: (i, 0)),
            pl.BlockSpec((_RT, 64), lambda i: (i, 0)),
        ],
        out_shape=[
            jax.ShapeDtypeStruct((N, 64), jnp.float32),
            jax.ShapeDtypeStruct((N, 64), jnp.float32),
        ],
    )(hflat, wp, bias, wq)


def _l0stats_body(a_ref, b_ref, t_ref, deg_ref, odeg_ref, g_ref, mu_ref, s_ref):
    i = pl.program_id(0)
    a = a_ref[...]
    b = b_ref[...]
    t = t_ref[...]
    deg = deg_ref[...]
    odeg = odeg_ref[...]

    @pl.when(i == 0)
    def _():
        mu_ref[...] = jnp.zeros_like(mu_ref)
        s_ref[...] = jnp.zeros_like(s_ref)

    mu_ref[...] += jnp.sum(deg * a + odeg * b, axis=0, keepdims=True)
    s_ref[...] += jnp.sum(deg * a * a + 2.0 * a * t + odeg * b * b,
                          axis=0, keepdims=True)

    @pl.when(i == pl.num_programs(0) - 1)
    def _():
        esz = jnp.float32(E0)
        mu = mu_ref[...] / esz
        var = s_ref[...] / esz - mu * mu
        mu_ref[...] = mu
        s_ref[...] = g_ref[...] * lax.rsqrt(var + 1e-5)


def _l0stats(a, b, t, deg, odeg, g):
    return pl.pallas_call(
        _l0stats_body,
        grid=(N // _RT,),
        in_specs=[
            pl.BlockSpec((_RT, 64), lambda i: (i, 0)),
            pl.BlockSpec((_RT, 64), lambda i: (i, 0)),
            pl.BlockSpec((_RT, 64), lambda i: (i, 0)),
            pl.BlockSpec((_RT, 1), lambda i: (i, 0)),
            pl.BlockSpec((_RT, 1), lambda i: (i, 0)),
            pl.BlockSpec((1, 64), lambda i: (0, 0)),
        ],
        out_specs=[
            pl.BlockSpec((1, 64), lambda i: (0, 0)),
            pl.BlockSpec((1, 64), lambda i: (0, 0)),
        ],
        out_shape=[
            jax.ShapeDtypeStruct((1, 64), jnp.float32),
            jax.ShapeDtypeStruct((1, 64), jnp.float32),
        ],
    )(a, b, t, deg, odeg, g)


def _l0finish_body(a_ref, mx_ref, mn_ref, deg_ref, f_ref, mu_ref, s_ref,
                   beta_ref, h_ref, hen_ref):
    s = s_ref[...]
    deg = deg_ref[...]
    mq = jnp.where(s >= 0, mx_ref[...], mn_ref[...])
    hm = (a_ref[...] + mq - mu_ref[...]) * s + beta_ref[...]
    h_ref[...] = _lrelu(jnp.where(deg > 0, hm, 0.0), 0.2)
    hen_ref[...] = f_ref[...] / jnp.maximum(deg, 1.0)


def _l0finish(a, mx, mn, deg, f, mu, s, beta):
    return pl.pallas_call(
        _l0finish_body,
        grid=(N // _RT,),
        in_specs=[
            pl.BlockSpec((_RT, 64), lambda i: (i, 0)),
            pl.BlockSpec((_RT, 64), lambda i: (i, 0)),
            pl.BlockSpec((_RT, 64), lambda i: (i, 0)),
            pl.BlockSpec((_RT, 1), lambda i: (i, 0)),
            pl.BlockSpec((_RT, 1), lambda i: (i, 0)),
            pl.BlockSpec((1, 64), lambda i: (0, 0)),
            pl.BlockSpec((1, 64), lambda i: (0, 0)),
            pl.BlockSpec((1, 64), lambda i: (0, 0)),
        ],
        out_specs=[
            pl.BlockSpec((_RT, 64), lambda i: (i, 0)),
            pl.BlockSpec((_RT, 1), lambda i: (i, 0)),
        ],
        out_shape=[
            jax.ShapeDtypeStruct((N, 64), jnp.float32),
            jax.ShapeDtypeStruct((N, 1), jnp.float32),
        ],
    )(a, mx, mn, deg, f, mu, s, beta)


def _proj_body(hc_ref, *refs):
    wrefs, out_ref = refs[:-1], refs[-1]
    y = hc_ref[...]
    nlayers = len(wrefs) // 2
    for j in range(nlayers):
        y = jnp.dot(y, wrefs[2 * j][...], preferred_element_type=jnp.float32) \
            + wrefs[2 * j + 1][...]
        if j < nlayers - 1:
            y = _lrelu(y, -0.8)
    out_ref[...] = y


def _proj(hc, proj_t):
    wspecs, wargs = [], []
    for (w, b) in proj_t:
        wspecs += [pl.BlockSpec(w.shape, lambda i: (0, 0)),
                   pl.BlockSpec(b.shape, lambda i: (0, 0))]
        wargs += [w, b]
    return pl.pallas_call(
        _proj_body,
        grid=(N // _RT,),
        in_specs=[pl.BlockSpec((_RT, 3), lambda i: (i, 0))] + wspecs,
        out_specs=pl.BlockSpec((_RT, 1), lambda i: (i, 0)),
        out_shape=jax.ShapeDtypeStruct((N, 1), jnp.float32),
    )(hc, *wargs)


def _tpose(layers):
    return [(w.T, b.reshape(1, -1)) for (w, b) in layers]


def kernel(x, x_en, edge_index, x_idx, length, params):
    enflat = x_en.reshape(N, 1)
    hs = []

    p0 = params['layers'][0]
    tW, tb = p0['theta']
    pW, pb = p0['phi']
    a, b = _l0prep(x.reshape(N, IN_DIM), tW.T, (tb + pb).reshape(1, 64),
                   (pW - tW).T)
    src, dst = edge_index[0], edge_index[1]
    bs = jnp.take(b, src, axis=0)
    t_rows = jnp.zeros((N, 64), jnp.float32).at[dst].add(bs)
    mx0 = jax.ops.segment_max(bs, dst, num_segments=N)
    mn0 = jax.ops.segment_min(bs, dst, num_segments=N)
    deg = jnp.zeros((N,), jnp.float32).at[dst].add(1.0).reshape(N, 1)
    odeg = jnp.zeros((N,), jnp.float32).at[src].add(1.0).reshape(N, 1)
    mu0, s0 = _l0stats(a, b, t_rows, deg, odeg, p0['bn_g'].reshape(1, 64))
    en_s = jnp.take(enflat, src, axis=0)
    en_d = jnp.take(enflat, dst, axis=0)
    e_en0 = _edge_mlp(en_d - en_s, en_s, _tpose(p0['ten']), _tpose(p0['pen']))
    f0 = jnp.zeros((N, 1), jnp.float32).at[dst].add(e_en0)
    h, h_en = _l0finish(a, mx0, mn0, deg, f0, mu0, s0,
                        p0['bn_b'].reshape(1, 64))
    hs.append(h_en)

    for i in (1, 2):
        pi = params['layers'][i]
        tW, tb = pi['theta']
        pW, pb = pi['phi']
        h3 = h.reshape(B, M, 64)
        p, table = _prep(h, h_en, tW.T, (tb + pb).reshape(1, 64), (pW - tW).T)
        nbr = _knn_pallas(h3)[:, :, 1:].reshape(-1)
        rows = jnp.take(table, nbr, axis=0)
        rows3 = rows.reshape(N, KM1, TBL_W)
        mu, s, mx, mn = _stats(p, rows3, pi['bn_g'].reshape(1, 64))
        en_nbr = rows3[:, :, 64]
        xd = (h_en - en_nbr).reshape(E_KNN, 1)
        xs = en_nbr.reshape(E_KNN, 1)
        e_en = _edge_mlp(xd, xs, _tpose(pi['ten']), _tpose(pi['pen']))
        h, h_en = _finish(p, mx, mn, e_en.reshape(N, KM1), mu, s,
                          pi['bn_b'].reshape(1, 64))
        hs.append(h_en)

    hc = jnp.concatenate(hs, axis=1)
    out_flat = _proj(hc, _tpose(params['proj'])).reshape((N,))
    imgs = []
    off = jnp.zeros((), jnp.int32)
    for bi in range(length.shape[0]):
        L = length[bi]
        e = lax.dynamic_slice_in_dim(out_flat, off, M)
        idx = lax.dynamic_slice(x_idx, (off, jnp.zeros((), jnp.int32)), (M, 3))
        valid = jnp.arange(M) < L
        i0 = jnp.where(valid, idx[:, 0], 7)
        img = jnp.zeros((7, 64, 64), out_flat.dtype).at[i0, idx[:, 1], idx[:, 2]].set(e)
        imgs.append(img[None])
        off = off + L
    oi = jnp.concatenate(imgs, 0)
    return oi[:, 1:, :, :]


# speedup vs baseline: 3.6724x; 1.0388x over previous
import functools

import jax
import jax.numpy as jnp
from jax import lax
from jax.experimental import pallas as pl

B, M, K_NN = 2, 4096, 16
IN_DIM = 3
N = B * M
KM1 = K_NN - 1
E_KNN = N * KM1
E0 = N * K_NN
TBL_W = 80

_KNN_R = 256
_RT = 512
_ET = 8192


def _lrelu(x, s):
    return jnp.where(x >= 0, x, s * x)


def _knn_body(hr_ref, hc_ref, out_ref):
    hr = hr_ref[0]
    hc = hc_ref[0]
    x2r = jnp.sum(hr * hr, axis=1, keepdims=True)
    x2c = jnp.sum(hc * hc, axis=1).reshape(1, M)
    d = x2r + x2c - 2.0 * lax.dot_general(
        hr, hc, dimension_numbers=(((1,), (1,)), ((), ())),
        preferred_element_type=jnp.float32)
    iota = lax.broadcasted_iota(jnp.int32, (_KNN_R, M), 1)
    cols = []
    for _ in range(K_NN):
        m = jnp.min(d, axis=1, keepdims=True)
        idx = jnp.min(jnp.where(d == m, iota, M), axis=1)
        cols.append(idx)
        d = jnp.where(iota == idx[:, None], jnp.inf, d)
    out_ref[0] = jnp.stack(cols, axis=1) + pl.program_id(0) * M


def _knn_pallas(h):
    D = h.shape[-1]
    return pl.pallas_call(
        _knn_body,
        grid=(B, M // _KNN_R),
        in_specs=[
            pl.BlockSpec((1, _KNN_R, D), lambda b, r: (b, r, 0)),
            pl.BlockSpec((1, M, D), lambda b, r: (b, 0, 0)),
        ],
        out_specs=pl.BlockSpec((1, _KNN_R, K_NN), lambda b, r: (b, r, 0)),
        out_shape=jax.ShapeDtypeStruct((B, M, K_NN), jnp.int32),
    )(h, h)


def _prep_body(h_ref, en_ref, wp_ref, bias_ref, wq_ref, p_ref, tab_ref):
    h = h_ref[...]
    p_ref[...] = jnp.dot(h, wp_ref[...], preferred_element_type=jnp.float32) + bias_ref[...]
    q = jnp.dot(h, wq_ref[...], preferred_element_type=jnp.float32)
    pad = jnp.zeros((h.shape[0], TBL_W - 65), jnp.float32)
    tab_ref[...] = jnp.concatenate([q, en_ref[...], pad], axis=1)


def _prep(hflat, enflat, wp, bias, wq):
    d = hflat.shape[1]
    return pl.pallas_call(
        _prep_body,
        grid=(N // _RT,),
        in_specs=[
            pl.BlockSpec((_RT, d), lambda i: (i, 0)),
            pl.BlockSpec((_RT, 1), lambda i: (i, 0)),
            pl.BlockSpec((d, 64), lambda i: (0, 0)),
            pl.BlockSpec((1, 64), lambda i: (0, 0)),
            pl.BlockSpec((d, 64), lambda i: (0, 0)),
        ],
        out_specs=[
            pl.BlockSpec((_RT, 64), lambda i: (i, 0)),
            pl.BlockSpec((_RT, TBL_W), lambda i: (i, 0)),
        ],
        out_shape=[
            jax.ShapeDtypeStruct((N, 64), jnp.float32),
            jax.ShapeDtypeStruct((N, TBL_W), jnp.float32),
        ],
    )(hflat, enflat, wp, bias, wq)


def _stats_body(p_ref, rows_ref, g_ref, mu_ref, s_ref, mx_ref, mn_ref):
    i = pl.program_id(0)
    p = p_ref[...]
    q = rows_ref[..., :64]
    s1 = jnp.sum(q, axis=1)
    s2 = jnp.sum(q * q, axis=1)
    mx_ref[...] = jnp.max(q, axis=1)
    mn_ref[...] = jnp.min(q, axis=1)

    @pl.when(i == 0)
    def _():
        mu_ref[...] = jnp.zeros_like(mu_ref)
        s_ref[...] = jnp.zeros_like(s_ref)

    mu_ref[...] += jnp.sum(KM1 * p + s1, axis=0, keepdims=True)
    s_ref[...] += jnp.sum(KM1 * p * p + 2.0 * p * s1 + s2, axis=0, keepdims=True)

    @pl.when(i == pl.num_programs(0) - 1)
    def _():
        esz = jnp.float32(E_KNN)
        mu = mu_ref[...] / esz
        var = s_ref[...] / esz - mu * mu
        mu_ref[...] = mu
        s_ref[...] = g_ref[...] * lax.rsqrt(var + 1e-5)


def _stats(p, rows3, g):
    return pl.pallas_call(
        _stats_body,
        grid=(N // _RT,),
        in_specs=[
            pl.BlockSpec((_RT, 64), lambda i: (i, 0)),
            pl.BlockSpec((_RT, KM1, TBL_W), lambda i: (i, 0, 0)),
            pl.BlockSpec((1, 64), lambda i: (0, 0)),
        ],
        out_specs=[
            pl.BlockSpec((1, 64), lambda i: (0, 0)),
            pl.BlockSpec((1, 64), lambda i: (0, 0)),
            pl.BlockSpec((_RT, 64), lambda i: (i, 0)),
            pl.BlockSpec((_RT, 64), lambda i: (i, 0)),
        ],
        out_shape=[
            jax.ShapeDtypeStruct((1, 64), jnp.float32),
            jax.ShapeDtypeStruct((1, 64), jnp.float32),
            jax.ShapeDtypeStruct((N, 64), jnp.float32),
            jax.ShapeDtypeStruct((N, 64), jnp.float32),
        ],
    )(p, rows3, g)


def _finish_body(p_ref, mx_ref, mn_ref, een_ref, mu_ref, s_ref, beta_ref,
                 h_ref, hen_ref):
    s = s_ref[...]
    mq = jnp.where(s >= 0, mx_ref[...], mn_ref[...])
    hm = (p_ref[...] + mq - mu_ref[...]) * s + beta_ref[...]
    h_ref[...] = _lrelu(hm, 0.2)
    hen_ref[...] = jnp.mean(een_ref[...], axis=1, keepdims=True)


def _finish(p, mx, mn, een_r, mu, s, beta):
    return pl.pallas_call(
        _finish_body,
        grid=(N // _RT,),
        in_specs=[
            pl.BlockSpec((_RT, 64), lambda i: (i, 0)),
            pl.BlockSpec((_RT, 64), lambda i: (i, 0)),
            pl.BlockSpec((_RT, 64), lambda i: (i, 0)),
            pl.BlockSpec((_RT, KM1), lambda i: (i, 0)),
            pl.BlockSpec((1, 64), lambda i: (0, 0)),
            pl.BlockSpec((1, 64), lambda i: (0, 0)),
            pl.BlockSpec((1, 64), lambda i: (0, 0)),
        ],
        out_specs=[
            pl.BlockSpec((_RT, 64), lambda i: (i, 0)),
            pl.BlockSpec((_RT, 1), lambda i: (i, 0)),
        ],
        out_shape=[
            jax.ShapeDtypeStruct((N, 64), jnp.float32),
            jax.ShapeDtypeStruct((N, 1), jnp.float32),
        ],
    )(p, mx, mn, een_r, mu, s, beta)


def _edge_mlp_body(xd_ref, xs_ref, *refs):
    wrefs, out_ref = refs[:-1], refs[-1]
    w = [r[...] for r in wrefs]

    def mlp(x, ws):
        w1, b1, w2, b2, w3, b3, w4, b4 = ws
        y = _lrelu(x * w1 + b1, -0.5)
        y = _lrelu(jnp.dot(y, w2, preferred_element_type=jnp.float32) + b2, -0.5)
        y = _lrelu(jnp.dot(y, w3, preferred_element_type=jnp.float32) + b3, -0.5)
        return jnp.dot(y, w4, preferred_element_type=jnp.float32) + b4

    out_ref[...] = mlp(xd_ref[...], w[:8]) + mlp(xs_ref[...], w[8:])


def _edge_mlp(xd, xs, ten_t, pen_t):
    e = xd.shape[0]
    wspecs, wargs = [], []
    for (w, b) in ten_t + pen_t:
        wspecs += [pl.BlockSpec(w.shape, lambda i: (0, 0)),
                   pl.BlockSpec(b.shape, lambda i: (0, 0))]
        wargs += [w, b]
    return pl.pallas_call(
        _edge_mlp_body,
        grid=(e // _ET,),
        in_specs=[pl.BlockSpec((_ET, 1), lambda i: (i, 0)),
                  pl.BlockSpec((_ET, 1), lambda i: (i, 0))] + wspecs,
        out_specs=pl.BlockSpec((_ET, 1), lambda i: (i, 0)),
        out_shape=jax.ShapeDtypeStruct((e, 1), jnp.float32),
    )(xd, xs, *wargs)


def _l0prep_body(h_ref, wp_ref, bias_ref, wq_ref, a_ref, b_ref):
    h = h_ref[...]
    a_ref[...] = jnp.dot(h, wp_ref[...], preferred_element_type=jnp.float32) + bias_ref[...]
    b_ref[...] = jnp.dot(h, wq_ref[...], preferred_element_type=jnp.float32)


def _l0prep(hflat, wp, bias, wq):
    d = hflat.shape[1]
    return pl.pallas_call(
        _l0prep_body,
        grid=(N // _RT,),
        in_specs=[
            pl.BlockSpec((_RT, d), lambda i: (i, 0)),
            pl.BlockSpec((d, 64), lambda i: (0, 0)),
            pl.BlockSpec((1, 64), lambda i: (0, 0)),
            pl.BlockSpec((d, 64), lambda i: (0, 0)),
        ],
        out_specs=[
            pl.BlockSpec((_RT, 64), lambda i: (i, 0)),
            pl.BlockSpec((_RT, 64), lambda i: (i, 0)),
        ],
        out_shape=[
            jax.ShapeDtypeStruct((N, 64), jnp.float32),
            jax.ShapeDtypeStruct((N, 64), jnp.float32),
        ],
    )(hflat, wp, bias, wq)


def _l0stats_body(a_ref, b_ref, t_ref, deg_ref, odeg_ref, g_ref, mu_ref, s_ref):
    i = pl.program_id(0)
    a = a_ref[...]
    b = b_ref[...]
    t = t_ref[...]
    deg = deg_ref[...]
    odeg = odeg_ref[...]

    @pl.when(i == 0)
    def _():
        mu_ref[...] = jnp.zeros_like(mu_ref)
        s_ref[...] = jnp.zeros_like(s_ref)

    mu_ref[...] += jnp.sum(deg * a + odeg * b, axis=0, keepdims=True)
    s_ref[...] += jnp.sum(deg * a * a + 2.0 * a * t + odeg * b * b,
                          axis=0, keepdims=True)

    @pl.when(i == pl.num_programs(0) - 1)
    def _():
        esz = jnp.float32(E0)
        mu = mu_ref[...] / esz
        var = s_ref[...] / esz - mu * mu
        mu_ref[...] = mu
        s_ref[...] = g_ref[...] * lax.rsqrt(var + 1e-5)


def _l0stats(a, b, t, deg, odeg, g):
    return pl.pallas_call(
        _l0stats_body,
        grid=(N // _RT,),
        in_specs=[
            pl.BlockSpec((_RT, 64), lambda i: (i, 0)),
            pl.BlockSpec((_RT, 64), lambda i: (i, 0)),
            pl.BlockSpec((_RT, 64), lambda i: (i, 0)),
            pl.BlockSpec((_RT, 1), lambda i: (i, 0)),
            pl.BlockSpec((_RT, 1), lambda i: (i, 0)),
            pl.BlockSpec((1, 64), lambda i: (0, 0)),
        ],
        out_specs=[
            pl.BlockSpec((1, 64), lambda i: (0, 0)),
            pl.BlockSpec((1, 64), lambda i: (0, 0)),
        ],
        out_shape=[
            jax.ShapeDtypeStruct((1, 64), jnp.float32),
            jax.ShapeDtypeStruct((1, 64), jnp.float32),
        ],
    )(a, b, t, deg, odeg, g)


def _l0finish_body(a_ref, mx_ref, mn_ref, deg_ref, f_ref, mu_ref, s_ref,
                   beta_ref, h_ref, hen_ref):
    s = s_ref[...]
    deg = deg_ref[...]
    mq = jnp.where(s >= 0, mx_ref[...], mn_ref[...])
    hm = (a_ref[...] + mq - mu_ref[...]) * s + beta_ref[...]
    h_ref[...] = _lrelu(jnp.where(deg > 0, hm, 0.0), 0.2)
    hen_ref[...] = f_ref[...] / jnp.maximum(deg, 1.0)


def _l0finish(a, mx, mn, deg, f, mu, s, beta):
    return pl.pallas_call(
        _l0finish_body,
        grid=(N // _RT,),
        in_specs=[
            pl.BlockSpec((_RT, 64), lambda i: (i, 0)),
            pl.BlockSpec((_RT, 64), lambda i: (i, 0)),
            pl.BlockSpec((_RT, 64), lambda i: (i, 0)),
            pl.BlockSpec((_RT, 1), lambda i: (i, 0)),
            pl.BlockSpec((_RT, 1), lambda i: (i, 0)),
            pl.BlockSpec((1, 64), lambda i: (0, 0)),
            pl.BlockSpec((1, 64), lambda i: (0, 0)),
            pl.BlockSpec((1, 64), lambda i: (0, 0)),
        ],
        out_specs=[
            pl.BlockSpec((_RT, 64), lambda i: (i, 0)),
            pl.BlockSpec((_RT, 1), lambda i: (i, 0)),
        ],
        out_shape=[
            jax.ShapeDtypeStruct((N, 64), jnp.float32),
            jax.ShapeDtypeStruct((N, 1), jnp.float32),
        ],
    )(a, mx, mn, deg, f, mu, s, beta)


def _proj_body(hc_ref, *refs):
    wrefs, out_ref = refs[:-1], refs[-1]
    y = hc_ref[...]
    nlayers = len(wrefs) // 2
    for j in range(nlayers):
        y = jnp.dot(y, wrefs[2 * j][...], preferred_element_type=jnp.float32) \
            + wrefs[2 * j + 1][...]
        if j < nlayers - 1:
            y = _lrelu(y, -0.8)
    out_ref[...] = y


def _proj(hc, proj_t):
    wspecs, wargs = [], []
    for (w, b) in proj_t:
        wspecs += [pl.BlockSpec(w.shape, lambda i: (0, 0)),
                   pl.BlockSpec(b.shape, lambda i: (0, 0))]
        wargs += [w, b]
    return pl.pallas_call(
        _proj_body,
        grid=(N // _RT,),
        in_specs=[pl.BlockSpec((_RT, 3), lambda i: (i, 0))] + wspecs,
        out_specs=pl.BlockSpec((_RT, 1), lambda i: (i, 0)),
        out_shape=jax.ShapeDtypeStruct((N, 1), jnp.float32),
    )(hc, *wargs)


def _tpose(layers):
    return [(w.T, b.reshape(1, -1)) for (w, b) in layers]


def kernel(x, x_en, edge_index, x_idx, length, params):
    enflat = x_en.reshape(N, 1)
    hs = []

    p0 = params['layers'][0]
    tW, tb = p0['theta']
    pW, pb = p0['phi']
    a, b = _l0prep(x.reshape(N, IN_DIM), tW.T, (tb + pb).reshape(1, 64),
                   (pW - tW).T)
    src, dst = edge_index[0], edge_index[1]
    bs = jnp.take(b, src, axis=0)
    en_s = jnp.take(enflat, src, axis=0)
    en_d = jnp.take(enflat, dst, axis=0)
    e_en0 = _edge_mlp(en_d - en_s, en_s, _tpose(p0['ten']), _tpose(p0['pen']))
    mxmn = jax.ops.segment_max(jnp.concatenate([bs, -bs], axis=1), dst,
                               num_segments=N)
    mx0, mn0 = mxmn[:, :64], -mxmn[:, 64:]
    ones_e = jnp.ones((E0, 1), jnp.float32)
    sadd = jnp.zeros((N, 66), jnp.float32).at[dst].add(
        jnp.concatenate([bs, e_en0, ones_e], axis=1))
    t_rows, f0, deg = sadd[:, :64], sadd[:, 64:65], sadd[:, 65:66]
    odeg = jnp.zeros((N,), jnp.float32).at[src].add(1.0).reshape(N, 1)
    mu0, s0 = _l0stats(a, b, t_rows, deg, odeg, p0['bn_g'].reshape(1, 64))
    h, h_en = _l0finish(a, mx0, mn0, deg, f0, mu0, s0,
                        p0['bn_b'].reshape(1, 64))
    hs.append(h_en)

    for i in (1, 2):
        pi = params['layers'][i]
        tW, tb = pi['theta']
        pW, pb = pi['phi']
        h3 = h.reshape(B, M, 64)
        p, table = _prep(h, h_en, tW.T, (tb + pb).reshape(1, 64), (pW - tW).T)
        nbr = _knn_pallas(h3)[:, :, 1:].reshape(-1)
        rows = jnp.take(table, nbr, axis=0)
        rows3 = rows.reshape(N, KM1, TBL_W)
        mu, s, mx, mn = _stats(p, rows3, pi['bn_g'].reshape(1, 64))
        en_nbr = rows3[:, :, 64]
        xd = (h_en - en_nbr).reshape(E_KNN, 1)
        xs = en_nbr.reshape(E_KNN, 1)
        e_en = _edge_mlp(xd, xs, _tpose(pi['ten']), _tpose(pi['pen']))
        h, h_en = _finish(p, mx, mn, e_en.reshape(N, KM1), mu, s,
                          pi['bn_b'].reshape(1, 64))
        hs.append(h_en)

    hc = jnp.concatenate(hs, axis=1)
    out_flat = _proj(hc, _tpose(params['proj'])).reshape((N,))
    imgs = []
    off = jnp.zeros((), jnp.int32)
    for bi in range(length.shape[0]):
        L = length[bi]
        e = lax.dynamic_slice_in_dim(out_flat, off, M)
        idx = lax.dynamic_slice(x_idx, (off, jnp.zeros((), jnp.int32)), (M, 3))
        valid = jnp.arange(M) < L
        i0 = jnp.where(valid, idx[:, 0], 7)
        img = jnp.zeros((7, 64, 64), out_flat.dtype).at[i0, idx[:, 1], idx[:, 2]].set(e)
        imgs.append(img[None])
        off = off + L
    oi = jnp.concatenate(imgs, 0)
    return oi[:, 1:, :, :]


# speedup vs baseline: 4.2010x; 1.1439x over previous
import functools

import jax
import jax.numpy as jnp
from jax import lax
from jax.experimental import pallas as pl

B, M, K_NN = 2, 4096, 16
IN_DIM = 3
N = B * M
KM1 = K_NN - 1
E_KNN = N * KM1
E0 = N * K_NN
TBL_W = 72

_KNN_R = 256
_RT = 512
_ET = 8192


def _lrelu(x, s):
    return jnp.where(x >= 0, x, s * x)


def _knn_body(hr_ref, hc_ref, out_ref):
    hr = hr_ref[0]
    hc = hc_ref[0]
    x2r = jnp.sum(hr * hr, axis=1, keepdims=True)
    x2c = jnp.sum(hc * hc, axis=1).reshape(1, M)
    d = x2r + x2c - 2.0 * lax.dot_general(
        hr, hc, dimension_numbers=(((1,), (1,)), ((), ())),
        preferred_element_type=jnp.float32)
    iota = lax.broadcasted_iota(jnp.int32, (_KNN_R, M), 1)
    cols = []
    for _ in range(K_NN):
        m = jnp.min(d, axis=1, keepdims=True)
        idx = jnp.min(jnp.where(d == m, iota, M), axis=1)
        cols.append(idx)
        d = jnp.where(iota == idx[:, None], jnp.inf, d)
    out_ref[0] = jnp.stack(cols, axis=1) + pl.program_id(0) * M


def _knn_pallas(h):
    D = h.shape[-1]
    return pl.pallas_call(
        _knn_body,
        grid=(B, M // _KNN_R),
        in_specs=[
            pl.BlockSpec((1, _KNN_R, D), lambda b, r: (b, r, 0)),
            pl.BlockSpec((1, M, D), lambda b, r: (b, 0, 0)),
        ],
        out_specs=pl.BlockSpec((1, _KNN_R, K_NN), lambda b, r: (b, r, 0)),
        out_shape=jax.ShapeDtypeStruct((B, M, K_NN), jnp.int32),
    )(h, h)


def _prep_body(h_ref, en_ref, wp_ref, bias_ref, wq_ref, p_ref, tab_ref):
    h = h_ref[...]
    p_ref[...] = jnp.dot(h, wp_ref[...], preferred_element_type=jnp.float32) + bias_ref[...]
    q = jnp.dot(h, wq_ref[...], preferred_element_type=jnp.float32)
    pad = jnp.zeros((h.shape[0], TBL_W - 65), jnp.float32)
    tab_ref[...] = jnp.concatenate([q, en_ref[...], pad], axis=1)


def _prep(hflat, enflat, wp, bias, wq):
    d = hflat.shape[1]
    return pl.pallas_call(
        _prep_body,
        grid=(N // _RT,),
        in_specs=[
            pl.BlockSpec((_RT, d), lambda i: (i, 0)),
            pl.BlockSpec((_RT, 1), lambda i: (i, 0)),
            pl.BlockSpec((d, 64), lambda i: (0, 0)),
            pl.BlockSpec((1, 64), lambda i: (0, 0)),
            pl.BlockSpec((d, 64), lambda i: (0, 0)),
        ],
        out_specs=[
            pl.BlockSpec((_RT, 64), lambda i: (i, 0)),
            pl.BlockSpec((_RT, TBL_W), lambda i: (i, 0)),
        ],
        out_shape=[
            jax.ShapeDtypeStruct((N, 64), jnp.float32),
            jax.ShapeDtypeStruct((N, TBL_W), jnp.float32),
        ],
    )(hflat, enflat, wp, bias, wq)


def _stats_body(p_ref, rows_ref, g_ref, mu_ref, s_ref, mx_ref, mn_ref):
    i = pl.program_id(0)
    p = p_ref[...]
    q = rows_ref[..., :64]
    s1 = jnp.sum(q, axis=1)
    s2 = jnp.sum(q * q, axis=1)
    mx_ref[...] = jnp.max(q, axis=1)
    mn_ref[...] = jnp.min(q, axis=1)

    @pl.when(i == 0)
    def _():
        mu_ref[...] = jnp.zeros_like(mu_ref)
        s_ref[...] = jnp.zeros_like(s_ref)

    mu_ref[...] += jnp.sum(KM1 * p + s1, axis=0, keepdims=True)
    s_ref[...] += jnp.sum(KM1 * p * p + 2.0 * p * s1 + s2, axis=0, keepdims=True)

    @pl.when(i == pl.num_programs(0) - 1)
    def _():
        esz = jnp.float32(E_KNN)
        mu = mu_ref[...] / esz
        var = s_ref[...] / esz - mu * mu
        mu_ref[...] = mu
        s_ref[...] = g_ref[...] * lax.rsqrt(var + 1e-5)


def _stats(p, rows3, g):
    return pl.pallas_call(
        _stats_body,
        grid=(N // _RT,),
        in_specs=[
            pl.BlockSpec((_RT, 64), lambda i: (i, 0)),
            pl.BlockSpec((_RT, KM1, TBL_W), lambda i: (i, 0, 0)),
            pl.BlockSpec((1, 64), lambda i: (0, 0)),
        ],
        out_specs=[
            pl.BlockSpec((1, 64), lambda i: (0, 0)),
            pl.BlockSpec((1, 64), lambda i: (0, 0)),
            pl.BlockSpec((_RT, 64), lambda i: (i, 0)),
            pl.BlockSpec((_RT, 64), lambda i: (i, 0)),
        ],
        out_shape=[
            jax.ShapeDtypeStruct((1, 64), jnp.float32),
            jax.ShapeDtypeStruct((1, 64), jnp.float32),
            jax.ShapeDtypeStruct((N, 64), jnp.float32),
            jax.ShapeDtypeStruct((N, 64), jnp.float32),
        ],
    )(p, rows3, g)


def _finish_body(p_ref, mx_ref, mn_ref, een_ref, mu_ref, s_ref, beta_ref,
                 h_ref, hen_ref):
    s = s_ref[...]
    mq = jnp.where(s >= 0, mx_ref[...], mn_ref[...])
    hm = (p_ref[...] + mq - mu_ref[...]) * s + beta_ref[...]
    h_ref[...] = _lrelu(hm, 0.2)
    hen_ref[...] = jnp.mean(een_ref[...], axis=1, keepdims=True)


def _finish(p, mx, mn, een_r, mu, s, beta):
    return pl.pallas_call(
        _finish_body,
        grid=(N // _RT,),
        in_specs=[
            pl.BlockSpec((_RT, 64), lambda i: (i, 0)),
            pl.BlockSpec((_RT, 64), lambda i: (i, 0)),
            pl.BlockSpec((_RT, 64), lambda i: (i, 0)),
            pl.BlockSpec((_RT, KM1), lambda i: (i, 0)),
            pl.BlockSpec((1, 64), lambda i: (0, 0)),
            pl.BlockSpec((1, 64), lambda i: (0, 0)),
            pl.BlockSpec((1, 64), lambda i: (0, 0)),
        ],
        out_specs=[
            pl.BlockSpec((_RT, 64), lambda i: (i, 0)),
            pl.BlockSpec((_RT, 1), lambda i: (i, 0)),
        ],
        out_shape=[
            jax.ShapeDtypeStruct((N, 64), jnp.float32),
            jax.ShapeDtypeStruct((N, 1), jnp.float32),
        ],
    )(p, mx, mn, een_r, mu, s, beta)


def _edge_mlp_body(xd_ref, xs_ref, *refs):
    wrefs, out_ref = refs[:-1], refs[-1]
    w = [r[...] for r in wrefs]

    def mlp(x, ws):
        w1, b1, w2, b2, w3, b3, w4, b4 = ws
        y = _lrelu(x * w1 + b1, -0.5)
        y = _lrelu(jnp.dot(y, w2, preferred_element_type=jnp.float32) + b2, -0.5)
        y = _lrelu(jnp.dot(y, w3, preferred_element_type=jnp.float32) + b3, -0.5)
        return jnp.dot(y, w4, preferred_element_type=jnp.float32) + b4

    out_ref[...] = mlp(xd_ref[...], w[:8]) + mlp(xs_ref[...], w[8:])


def _edge_mlp(xd, xs, ten_t, pen_t):
    e = xd.shape[0]
    wspecs, wargs = [], []
    for (w, b) in ten_t + pen_t:
        wspecs += [pl.BlockSpec(w.shape, lambda i: (0, 0)),
                   pl.BlockSpec(b.shape, lambda i: (0, 0))]
        wargs += [w, b]
    return pl.pallas_call(
        _edge_mlp_body,
        grid=(e // _ET,),
        in_specs=[pl.BlockSpec((_ET, 1), lambda i: (i, 0)),
                  pl.BlockSpec((_ET, 1), lambda i: (i, 0))] + wspecs,
        out_specs=pl.BlockSpec((_ET, 1), lambda i: (i, 0)),
        out_shape=jax.ShapeDtypeStruct((e, 1), jnp.float32),
    )(xd, xs, *wargs)


def _l0prep_body(h_ref, wp_ref, bias_ref, wq_ref, a_ref, b_ref):
    h = h_ref[...]
    a_ref[...] = jnp.dot(h, wp_ref[...], preferred_element_type=jnp.float32) + bias_ref[...]
    b_ref[...] = jnp.dot(h, wq_ref[...], preferred_element_type=jnp.float32)


def _l0prep(hflat, wp, bias, wq):
    d = hflat.shape[1]
    return pl.pallas_call(
        _l0prep_body,
        grid=(N // _RT,),
        in_specs=[
            pl.BlockSpec((_RT, d), lambda i: (i, 0)),
            pl.BlockSpec((d, 64), lambda i: (0, 0)),
            pl.BlockSpec((1, 64), lambda i: (0, 0)),
            pl.BlockSpec((d, 64), lambda i: (0, 0)),
        ],
        out_specs=[
            pl.BlockSpec((_RT, 64), lambda i: (i, 0)),
            pl.BlockSpec((_RT, 64), lambda i: (i, 0)),
        ],
        out_shape=[
            jax.ShapeDtypeStruct((N, 64), jnp.float32),
            jax.ShapeDtypeStruct((N, 64), jnp.float32),
        ],
    )(hflat, wp, bias, wq)


def _l0stats_body(a_ref, b_ref, t_ref, deg_ref, odeg_ref, g_ref, mu_ref, s_ref):
    i = pl.program_id(0)
    a = a_ref[...]
    b = b_ref[...]
    t = t_ref[...]
    deg = deg_ref[...]
    odeg = odeg_ref[...]

    @pl.when(i == 0)
    def _():
        mu_ref[...] = jnp.zeros_like(mu_ref)
        s_ref[...] = jnp.zeros_like(s_ref)

    mu_ref[...] += jnp.sum(deg * a + odeg * b, axis=0, keepdims=True)
    s_ref[...] += jnp.sum(deg * a * a + 2.0 * a * t + odeg * b * b,
                          axis=0, keepdims=True)

    @pl.when(i == pl.num_programs(0) - 1)
    def _():
        esz = jnp.float32(E0)
        mu = mu_ref[...] / esz
        var = s_ref[...] / esz - mu * mu
        mu_ref[...] = mu
        s_ref[...] = g_ref[...] * lax.rsqrt(var + 1e-5)


def _l0stats(a, b, t, deg, odeg, g):
    return pl.pallas_call(
        _l0stats_body,
        grid=(N // _RT,),
        in_specs=[
            pl.BlockSpec((_RT, 64), lambda i: (i, 0)),
            pl.BlockSpec((_RT, 64), lambda i: (i, 0)),
            pl.BlockSpec((_RT, 64), lambda i: (i, 0)),
            pl.BlockSpec((_RT, 1), lambda i: (i, 0)),
            pl.BlockSpec((_RT, 1), lambda i: (i, 0)),
            pl.BlockSpec((1, 64), lambda i: (0, 0)),
        ],
        out_specs=[
            pl.BlockSpec((1, 64), lambda i: (0, 0)),
            pl.BlockSpec((1, 64), lambda i: (0, 0)),
        ],
        out_shape=[
            jax.ShapeDtypeStruct((1, 64), jnp.float32),
            jax.ShapeDtypeStruct((1, 64), jnp.float32),
        ],
    )(a, b, t, deg, odeg, g)


def _l0finish_body(a_ref, sm_ref, deg_ref, f_ref, mu_ref, s_ref,
                   beta_ref, h_ref, hen_ref):
    s = s_ref[...]
    deg = deg_ref[...]
    mq = jnp.abs(s) * sm_ref[...]
    hm = (a_ref[...] - mu_ref[...]) * s + mq + beta_ref[...]
    h_ref[...] = _lrelu(jnp.where(deg > 0, hm, 0.0), 0.2)
    hen_ref[...] = f_ref[...] / jnp.maximum(deg, 1.0)


def _l0finish(a, sm, deg, f, mu, s, beta):
    return pl.pallas_call(
        _l0finish_body,
        grid=(N // _RT,),
        in_specs=[
            pl.BlockSpec((_RT, 64), lambda i: (i, 0)),
            pl.BlockSpec((_RT, 64), lambda i: (i, 0)),
            pl.BlockSpec((_RT, 1), lambda i: (i, 0)),
            pl.BlockSpec((_RT, 1), lambda i: (i, 0)),
            pl.BlockSpec((1, 64), lambda i: (0, 0)),
            pl.BlockSpec((1, 64), lambda i: (0, 0)),
            pl.BlockSpec((1, 64), lambda i: (0, 0)),
        ],
        out_specs=[
            pl.BlockSpec((_RT, 64), lambda i: (i, 0)),
            pl.BlockSpec((_RT, 1), lambda i: (i, 0)),
        ],
        out_shape=[
            jax.ShapeDtypeStruct((N, 64), jnp.float32),
            jax.ShapeDtypeStruct((N, 1), jnp.float32),
        ],
    )(a, sm, deg, f, mu, s, beta)


def _proj_body(hc_ref, *refs):
    wrefs, out_ref = refs[:-1], refs[-1]
    y = hc_ref[...]
    nlayers = len(wrefs) // 2
    for j in range(nlayers):
        y = jnp.dot(y, wrefs[2 * j][...], preferred_element_type=jnp.float32) \
            + wrefs[2 * j + 1][...]
        if j < nlayers - 1:
            y = _lrelu(y, -0.8)
    out_ref[...] = y


def _proj(hc, proj_t):
    wspecs, wargs = [], []
    for (w, b) in proj_t:
        wspecs += [pl.BlockSpec(w.shape, lambda i: (0, 0)),
                   pl.BlockSpec(b.shape, lambda i: (0, 0))]
        wargs += [w, b]
    return pl.pallas_call(
        _proj_body,
        grid=(N // _RT,),
        in_specs=[pl.BlockSpec((_RT, 3), lambda i: (i, 0))] + wspecs,
        out_specs=pl.BlockSpec((_RT, 1), lambda i: (i, 0)),
        out_shape=jax.ShapeDtypeStruct((N, 1), jnp.float32),
    )(hc, *wargs)


def _tpose(layers):
    return [(w.T, b.reshape(1, -1)) for (w, b) in layers]


def kernel(x, x_en, edge_index, x_idx, length, params):
    enflat = x_en.reshape(N, 1)
    hs = []

    p0 = params['layers'][0]
    tW, tb = p0['theta']
    pW, pb = p0['phi']
    a, b = _l0prep(x.reshape(N, IN_DIM), tW.T, (tb + pb).reshape(1, 64),
                   (pW - tW).T)
    src, dst = edge_index[0], edge_index[1]
    bsrows = jnp.take(jnp.concatenate([b, enflat], axis=1), src, axis=0)
    bs, en_s = bsrows[:, :64], bsrows[:, 64:65]
    en_d = jnp.take(enflat, dst, axis=0)
    e_en0 = _edge_mlp(en_d - en_s, en_s, _tpose(p0['ten']), _tpose(p0['pen']))
    ones_e = jnp.ones((E0, 1), jnp.float32)
    sadd = jnp.zeros((N, 66), jnp.float32).at[dst].add(
        jnp.concatenate([bs, e_en0, ones_e], axis=1))
    t_rows, f0, deg = sadd[:, :64], sadd[:, 64:65], sadd[:, 65:66]
    odeg = jnp.zeros((N,), jnp.float32).at[src].add(1.0).reshape(N, 1)
    mu0, s0 = _l0stats(a, b, t_rows, deg, odeg, p0['bn_g'].reshape(1, 64))
    sm0 = jax.ops.segment_max(jnp.where(s0 >= 0, bs, -bs), dst,
                              num_segments=N)
    h, h_en = _l0finish(a, sm0, deg, f0, mu0, s0,
                        p0['bn_b'].reshape(1, 64))
    hs.append(h_en)

    for i in (1, 2):
        pi = params['layers'][i]
        tW, tb = pi['theta']
        pW, pb = pi['phi']
        h3 = h.reshape(B, M, 64)
        p, table = _prep(h, h_en, tW.T, (tb + pb).reshape(1, 64), (pW - tW).T)
        nbr = _knn_pallas(h3)[:, :, 1:].reshape(-1)
        rows = jnp.take(table, nbr, axis=0)
        rows3 = rows.reshape(N, KM1, TBL_W)
        mu, s, mx, mn = _stats(p, rows3, pi['bn_g'].reshape(1, 64))
        en_nbr = rows3[:, :, 64]
        xd = (h_en - en_nbr).reshape(E_KNN, 1)
        xs = en_nbr.reshape(E_KNN, 1)
        e_en = _edge_mlp(xd, xs, _tpose(pi['ten']), _tpose(pi['pen']))
        h, h_en = _finish(p, mx, mn, e_en.reshape(N, KM1), mu, s,
                          pi['bn_b'].reshape(1, 64))
        hs.append(h_en)

    hc = jnp.concatenate(hs, axis=1)
    out_flat = _proj(hc, _tpose(params['proj'])).reshape((N,))
    imgs = []
    off = jnp.zeros((), jnp.int32)
    for bi in range(length.shape[0]):
        L = length[bi]
        e = lax.dynamic_slice_in_dim(out_flat, off, M)
        idx = lax.dynamic_slice(x_idx, (off, jnp.zeros((), jnp.int32)), (M, 3))
        valid = jnp.arange(M) < L
        i0 = jnp.where(valid, idx[:, 0], 7)
        img = jnp.zeros((7, 64, 64), out_flat.dtype).at[i0, idx[:, 1], idx[:, 2]].set(e)
        imgs.append(img[None])
        off = off + L
    oi = jnp.concatenate(imgs, 0)
    return oi[:, 1:, :, :]
